# baseline (speedup 1.0000x reference)
import functools

import jax
import jax.numpy as jnp
from jax import lax
from jax.experimental import pallas as pl
from jax.experimental.pallas import tpu as pltpu
from jax.experimental.pallas import tpu_sc as plsc

NUM_E = 8192
DIM = 32
ROWS = 8192
ROW_TILE = 1024
N_TILES = ROWS // ROW_TILE
COMMIT_W = 0.25


N_CHUNK = 8
CHUNK = NUM_E // N_CHUNK


def _dist_argmin_body(flat_ref, dith2_ref, x2_ref, c2_ref, out_ref, d2_scr):
    flat = flat_ref[...]
    x2 = x2_ref[...]
    rmin = jnp.full((ROW_TILE, 128), jnp.inf, jnp.float32)
    for j in range(N_CHUNK):
        mm = lax.dot_general(
            flat, dith2_ref[j * CHUNK:(j + 1) * CHUNK, :],
            (((1,), (1,)), ((), ())),
            preferred_element_type=jnp.float32,
        )
        d2 = (x2 + c2_ref[:, j * CHUNK:(j + 1) * CHUNK]) - 2.0 * mm
        dist = jnp.sqrt(jnp.maximum(d2, 0.0))
        d2_scr[:, j * CHUNK:(j + 1) * CHUNK] = dist
        for a in range(CHUNK // 128):
            rmin = jnp.minimum(rmin, dist[:, a * 128:(a + 1) * 128])
    m2 = jnp.min(rmin, axis=1, keepdims=True)
    sb = jnp.broadcast_to(m2, (ROW_TILE, 128))
    racc = jnp.full((ROW_TILE, 128), 1e4, jnp.float32)
    for j in reversed(range(N_CHUNK)):
        dist = d2_scr[:, j * CHUNK:(j + 1) * CHUNK]
        for a in reversed(range(CHUNK // 128)):
            t = jnp.float32(j * (CHUNK // 128) + a)
            cond = dist[:, a * 128:(a + 1) * 128] == sb
            racc = jnp.where(cond, t, racc)
    base = lax.broadcasted_iota(jnp.int32, (ROW_TILE, 128), 1).astype(jnp.float32)
    idxf = jnp.min(racc * 128.0 + base, axis=1)
    out_ref[0, 0, :] = idxf.astype(jnp.int32)


def _dist_argmin(flat, dith2_p, x2, c2_p):
    return pl.pallas_call(
        _dist_argmin_body,
        grid=(N_TILES,),
        in_specs=[
            pl.BlockSpec((ROW_TILE, DIM), lambda i: (i, 0)),
            pl.BlockSpec((NUM_E, DIM), lambda i: (0, 0)),
            pl.BlockSpec((ROW_TILE, 1), lambda i: (i, 0)),
            pl.BlockSpec((1, NUM_E), lambda i: (0, 0)),
        ],
        out_specs=pl.BlockSpec((1, 1, ROW_TILE), lambda i: (i, 0, 0)),
        out_shape=jax.ShapeDtypeStruct((N_TILES, 1, ROW_TILE), jnp.int32),
        scratch_shapes=[pltpu.VMEM((ROW_TILE, NUM_E), jnp.float32)],
    )(flat, dith2_p, x2, c2_p)


_SC_NC = 2
_SC_NS = 16
_SC_NW = _SC_NC * _SC_NS
_SC_RPW = ROWS // _SC_NW // 128


def _sc_gather_body(idx_hbm, table_hbm, zq_hbm, idx_v, rows_v, sem):
    wid = lax.axis_index("s") * _SC_NC + lax.axis_index("c")
    base = wid * _SC_RPW
    pltpu.sync_copy(idx_hbm.at[pl.ds(base, _SC_RPW)], idx_v)
    copies = [
        pltpu.async_copy(table_hbm.at[idx_v.at[k]], rows_v.at[k], sem)
        for k in range(_SC_RPW)
    ]
    for c in copies:
        c.wait()
    pltpu.sync_copy(rows_v, zq_hbm.at[pl.ds(base, _SC_RPW)])


def _sc_gather(idx2, table):
    mesh = plsc.VectorSubcoreMesh(core_axis_name="c", subcore_axis_name="s")
    f = pl.kernel(
        _sc_gather_body, mesh=mesh,
        compiler_params=pltpu.CompilerParams(use_tc_tiling_on_sc=False),
        out_type=jax.ShapeDtypeStruct((ROWS // 128, 128, DIM), jnp.float32),
        scratch_types=[
            pltpu.VMEM((_SC_RPW, 128), jnp.int32),
            pltpu.VMEM((_SC_RPW, 128, DIM), jnp.float32),
            pltpu.SemaphoreType.DMA,
        ],
    )
    return f(idx2, table)


def kernel(z, codebook, lambda_vals):
    input_shape = z.shape
    flat = z.reshape(-1, DIM)
    c_i = codebook[:-1]
    c_i_plus_1 = codebook[1:]
    dith = (1.0 - lambda_vals) * c_i + lambda_vals * c_i_plus_1
    x2 = jnp.sum(flat * flat, axis=1, keepdims=True)
    c2 = jnp.sum(dith * dith, axis=1)
    dith_p = jnp.concatenate([dith, jnp.zeros((1, DIM), jnp.float32)], axis=0)
    c2_p = jnp.concatenate([c2, jnp.full((1,), 1e30, jnp.float32)])[None, :]

    idx = _dist_argmin(flat, dith_p, x2, c2_p).reshape(-1)

    dith_g = _sc_gather(idx.reshape(ROWS // 128, 128), dith_p).reshape(ROWS, DIM)
    z_q = dith_g.reshape(input_shape)
    m = jnp.mean((dith_g - flat) ** 2)
    loss = m + COMMIT_W * m
    idx_out = idx.reshape(input_shape[:-1])
    return (z_q, loss, idx_out)


# speedup vs baseline: 1.0380x; 1.0380x over previous
import functools

import jax
import jax.numpy as jnp
from jax import lax
from jax.experimental import pallas as pl
from jax.experimental.pallas import tpu as pltpu
from jax.experimental.pallas import tpu_sc as plsc

NUM_E = 8192
DIM = 32
ROWS = 8192
ROW_TILE = 1024
N_TILES = ROWS // ROW_TILE
COMMIT_W = 0.25


N_CHUNK = 8
CHUNK = NUM_E // N_CHUNK


def _dist_argmin_body(flat_ref, dith_ref, x2_ref, c2_ref, out_ref, d2_scr):
    flat = flat_ref[...]
    x2 = x2_ref[...]
    rmin = jnp.full((ROW_TILE, 128), jnp.inf, jnp.float32)
    for j in range(N_CHUNK):
        mm = lax.dot_general(
            flat, dith_ref[j * CHUNK:(j + 1) * CHUNK, :],
            (((1,), (1,)), ((), ())),
            preferred_element_type=jnp.float32,
        )
        d2 = (x2 + c2_ref[:, j * CHUNK:(j + 1) * CHUNK]) - mm
        dist = jnp.sqrt(jnp.maximum(d2, 0.0))
        d2_scr[:, j * CHUNK:(j + 1) * CHUNK] = dist
        for a in range(CHUNK // 128):
            rmin = jnp.minimum(rmin, dist[:, a * 128:(a + 1) * 128])
    m2 = jnp.min(rmin, axis=1, keepdims=True)
    sb = jnp.broadcast_to(m2, (ROW_TILE, 128))
    racc = jnp.full((ROW_TILE, 128), 1e4, jnp.float32)
    for j in reversed(range(N_CHUNK)):
        dist = d2_scr[:, j * CHUNK:(j + 1) * CHUNK]
        for a in reversed(range(CHUNK // 128)):
            t = jnp.float32(j * (CHUNK // 128) + a)
            cond = dist[:, a * 128:(a + 1) * 128] == sb
            racc = jnp.where(cond, t, racc)
    base = lax.broadcasted_iota(jnp.int32, (ROW_TILE, 128), 1).astype(jnp.float32)
    idxf = jnp.min(racc * 128.0 + base, axis=1)
    out_ref[0, 0, :] = idxf.astype(jnp.int32)


def _dist_argmin(flat, dith_p, x2, c2_p):
    return pl.pallas_call(
        _dist_argmin_body,
        grid=(N_TILES,),
        in_specs=[
            pl.BlockSpec((ROW_TILE, DIM), lambda i: (i, 0)),
            pl.BlockSpec((NUM_E, DIM), lambda i: (0, 0)),
            pl.BlockSpec((ROW_TILE, 1), lambda i: (i, 0)),
            pl.BlockSpec((1, NUM_E), lambda i: (0, 0)),
        ],
        out_specs=pl.BlockSpec((1, 1, ROW_TILE), lambda i: (i, 0, 0)),
        out_shape=jax.ShapeDtypeStruct((N_TILES, 1, ROW_TILE), jnp.int32),
        scratch_shapes=[pltpu.VMEM((ROW_TILE, NUM_E), jnp.float32)],
    )(flat, dith_p, x2, c2_p)


_SC_NC = 2
_SC_NS = 16
_SC_NW = _SC_NC * _SC_NS
_SC_RPW = ROWS // _SC_NW // 128


def _sc_gather_body(idx_hbm, table_hbm, zq_hbm, idx_v, rows_v, sem):
    wid = lax.axis_index("s") * _SC_NC + lax.axis_index("c")
    base = wid * _SC_RPW
    pltpu.sync_copy(idx_hbm.at[pl.ds(base, _SC_RPW)], idx_v)
    copies = [
        pltpu.async_copy(table_hbm.at[idx_v.at[k]], rows_v.at[k], sem)
        for k in range(_SC_RPW)
    ]
    for c in copies:
        c.wait()
    pltpu.sync_copy(rows_v, zq_hbm.at[pl.ds(base, _SC_RPW)])


def _sc_gather(idx2, table):
    mesh = plsc.VectorSubcoreMesh(core_axis_name="c", subcore_axis_name="s")
    f = pl.kernel(
        _sc_gather_body, mesh=mesh,
        compiler_params=pltpu.CompilerParams(use_tc_tiling_on_sc=False),
        out_type=jax.ShapeDtypeStruct((ROWS // 128, 128, DIM), jnp.float32),
        scratch_types=[
            pltpu.VMEM((_SC_RPW, 128), jnp.int32),
            pltpu.VMEM((_SC_RPW, 128, DIM), jnp.float32),
            pltpu.SemaphoreType.DMA,
        ],
    )
    return f(idx2, table)


def kernel(z, codebook, lambda_vals):
    input_shape = z.shape
    flat = z.reshape(-1, DIM)
    c_i = codebook[:-1]
    c_i_plus_1 = codebook[1:]
    dith = (1.0 - lambda_vals) * c_i + lambda_vals * c_i_plus_1
    x2 = jnp.sum(flat * flat, axis=1, keepdims=True)
    c2 = jnp.sum(dith * dith, axis=1)
    dith_p = jnp.concatenate([dith, jnp.zeros((1, DIM), jnp.float32)], axis=0)
    c2_p = jnp.concatenate([c2, jnp.full((1,), 1e30, jnp.float32)])[None, :]

    idx = _dist_argmin(flat, dith_p * 2.0, x2, c2_p).reshape(-1)

    dith_g = _sc_gather(idx.reshape(ROWS // 128, 128), dith_p).reshape(ROWS, DIM)
    z_q = dith_g.reshape(input_shape)
    m = jnp.mean((dith_g - flat) ** 2)
    loss = m + COMMIT_W * m
    idx_out = idx.reshape(input_shape[:-1])
    return (z_q, loss, idx_out)


# speedup vs baseline: 1.0386x; 1.0005x over previous
import jax
import jax.numpy as jnp
from jax import lax
from jax.experimental import pallas as pl
from jax.experimental.pallas import tpu as pltpu
from jax.experimental.pallas import tpu_sc as plsc

NUM_E = 8192
DIM = 32
ROWS = 8192
ROW_TILE = 1024
N_TILES = ROWS // ROW_TILE
COMMIT_W = 0.25


N_CHUNK = 8
CHUNK = NUM_E // N_CHUNK


def _dist_argmin_body(flat_ref, dith2_ref, x2_ref, c2_ref, out_ref, d2_scr):
    flat = flat_ref[...]
    x2 = x2_ref[...]
    rmin = jnp.full((ROW_TILE, 128), jnp.inf, jnp.float32)
    for j in range(N_CHUNK):
        mm = lax.dot_general(
            flat, dith2_ref[j * CHUNK:(j + 1) * CHUNK, :],
            (((1,), (1,)), ((), ())),
            preferred_element_type=jnp.float32,
        )
        d2 = (x2 + c2_ref[:, j * CHUNK:(j + 1) * CHUNK]) - mm
        dist = jnp.sqrt(jnp.maximum(d2, 0.0))
        d2_scr[:, j * CHUNK:(j + 1) * CHUNK] = dist
        for a in range(CHUNK // 128):
            rmin = jnp.minimum(rmin, dist[:, a * 128:(a + 1) * 128])
    m2 = jnp.min(rmin, axis=1, keepdims=True)
    sb = jnp.broadcast_to(m2, (ROW_TILE, 128))
    racc = jnp.full((ROW_TILE, 128), 1e4, jnp.float32)
    for j in reversed(range(N_CHUNK)):
        dist = d2_scr[:, j * CHUNK:(j + 1) * CHUNK]
        for a in reversed(range(CHUNK // 128)):
            t = jnp.float32(j * (CHUNK // 128) + a)
            cond = dist[:, a * 128:(a + 1) * 128] == sb
            racc = jnp.where(cond, t, racc)
    base = lax.broadcasted_iota(jnp.int32, (ROW_TILE, 128), 1).astype(jnp.float32)
    idxf = jnp.min(racc * 128.0 + base, axis=1)
    out_ref[0, 0, :] = idxf.astype(jnp.int32)


def _dist_argmin(flat, dith_p, x2, c2_p):
    return pl.pallas_call(
        _dist_argmin_body,
        grid=(N_TILES,),
        in_specs=[
            pl.BlockSpec((ROW_TILE, DIM), lambda i: (i, 0)),
            pl.BlockSpec((NUM_E, DIM), lambda i: (0, 0)),
            pl.BlockSpec((ROW_TILE, 1), lambda i: (i, 0)),
            pl.BlockSpec((1, NUM_E), lambda i: (0, 0)),
        ],
        out_specs=pl.BlockSpec((1, 1, ROW_TILE), lambda i: (i, 0, 0)),
        out_shape=jax.ShapeDtypeStruct((N_TILES, 1, ROW_TILE), jnp.int32),
        scratch_shapes=[pltpu.VMEM((ROW_TILE, NUM_E), jnp.float32)],
    )(flat, dith_p, x2, c2_p)


_SC_NC = 2
_SC_NS = 16
_SC_NW = _SC_NC * _SC_NS
_SC_RPW = ROWS // _SC_NW // 128


def _sc_gather_body(idx_hbm, table_hbm, zq_hbm, idx_v, rows_v, sem):
    wid = lax.axis_index("s") * _SC_NC + lax.axis_index("c")
    base = wid * _SC_RPW
    pltpu.sync_copy(idx_hbm.at[pl.ds(base, _SC_RPW)], idx_v)
    copies = [
        pltpu.async_copy(table_hbm.at[idx_v.at[k]], rows_v.at[k], sem)
        for k in range(_SC_RPW)
    ]
    for c in copies:
        c.wait()
    pltpu.sync_copy(rows_v, zq_hbm.at[pl.ds(base, _SC_RPW)])


def _sc_gather(idx2, table):
    mesh = plsc.VectorSubcoreMesh(core_axis_name="c", subcore_axis_name="s")
    f = pl.kernel(
        _sc_gather_body, mesh=mesh,
        compiler_params=pltpu.CompilerParams(use_tc_tiling_on_sc=False),
        out_type=jax.ShapeDtypeStruct((ROWS // 128, 128, DIM), jnp.float32),
        scratch_types=[
            pltpu.VMEM((_SC_RPW, 128), jnp.int32),
            pltpu.VMEM((_SC_RPW, 128, DIM), jnp.float32),
            pltpu.SemaphoreType.DMA,
        ],
    )
    return f(idx2, table)


def kernel(z, codebook, lambda_vals):
    input_shape = z.shape
    flat = z.reshape(-1, DIM)
    c_i = codebook[:-1]
    c_i_plus_1 = codebook[1:]
    dith = (1.0 - lambda_vals) * c_i + lambda_vals * c_i_plus_1
    x2 = jnp.sum(flat * flat, axis=1, keepdims=True)
    c2 = jnp.sum(dith * dith, axis=1)
    dith_p = jnp.concatenate([dith, jnp.zeros((1, DIM), jnp.float32)], axis=0)
    c2_p = jnp.concatenate([c2, jnp.full((1,), 1e30, jnp.float32)])[None, :]

    idx = _dist_argmin(flat, dith_p * 2.0, x2, c2_p).reshape(-1)

    dith_g = _sc_gather(idx.reshape(ROWS // 128, 128), dith_p).reshape(ROWS, DIM)
    z_q = dith_g.reshape(input_shape)
    m = jnp.mean((dith_g - flat) ** 2)
    loss = m + COMMIT_W * m
    idx_out = idx.reshape(input_shape[:-1])
    return (z_q, loss, idx_out)
